# Initial kernel scaffold; baseline (speedup 1.0000x reference)
import jax
import jax.numpy as jnp
from jax.experimental import pallas as pl


def kernel(query, c_keys, c_prime_keys, ln_weight, ln_bias):
    raise NotImplementedError("write your pallas kernel here")


# speedup vs baseline: 30.0356x; 30.0356x over previous
import functools

import jax
import jax.numpy as jnp
from jax.experimental import pallas as pl
from jax.experimental.pallas import tpu as pltpu

D = 256
SUB = 128
SQRT_N = 512
K = 8

NEG_INF = float("-inf")


def _top8(s, width):
    lane = jax.lax.broadcasted_iota(jnp.int32, s.shape, 1)
    cur = s
    vals = []
    idxs = []
    for _ in range(K):
        m = jnp.max(cur, axis=-1, keepdims=True)
        hit = cur == m
        j = jnp.min(jnp.where(hit, lane, width), axis=-1, keepdims=True)
        vals.append(m)
        idxs.append(j)
        cur = jnp.where(lane == j, NEG_INF, cur)
    return jnp.concatenate(vals, axis=-1), jnp.concatenate(idxs, axis=-1)


def _onehot_gather(tab, k):
    acc = jnp.zeros(k.shape, tab.dtype)
    for t in range(K):
        acc = acc + jnp.where(k == t, tab[:, t : t + 1], 0)
    return acc


def _body(q_ref, ck1_ref, ck2_ref, w_ref, b_ref,
          gidx_ref, fsc_ref, aux_ref, ps_ref, *, grid):
    i = pl.program_id(0)

    @pl.when(i == 0)
    def _init():
        ps_ref[...] = jnp.zeros_like(ps_ref)

    q = q_ref[...]
    mu = jnp.mean(q, axis=-1, keepdims=True)
    var = jnp.mean((q - mu) ** 2, axis=-1, keepdims=True)
    qn = (q - mu) / jnp.sqrt(var + 1e-5) * w_ref[...] + b_ref[...]

    s1 = jnp.dot(qn[:, :SUB], ck1_ref[...], preferred_element_type=jnp.float32)
    s2 = jnp.dot(qn[:, SUB:], ck2_ref[...], preferred_element_type=jnp.float32)

    e1 = jnp.exp(s1)
    e2 = jnp.exp(s2)
    p1 = e1 / jnp.sum(e1, axis=-1, keepdims=True)
    p2 = e2 / jnp.sum(e2, axis=-1, keepdims=True)
    ps_ref[0:1, :] = ps_ref[0:1, :] + jnp.sum(p1, axis=0, keepdims=True)
    ps_ref[1:2, :] = ps_ref[1:2, :] + jnp.sum(p2, axis=0, keepdims=True)

    v1, i1 = _top8(s1, SQRT_N)
    v2, i2 = _top8(s2, SQRT_N)

    joint = jnp.concatenate([v1[:, t : t + 1] + v2 for t in range(K)], axis=-1)
    jv, jf = _top8(joint, K * K)
    r = jf >> 3
    c = jf & 7
    real_row = _onehot_gather(i1, r)
    real_col = _onehot_gather(i2, c)
    gidx_ref[...] = real_row * SQRT_N + real_col
    fsc_ref[...] = jv

    @pl.when(i == grid - 1)
    def _fin():
        n_rows = grid * q_ref.shape[0]
        ps = ps_ref[...] * (1.0 / n_rows)
        aux_ref[...] = jnp.sum(ps * ps).reshape(1, 1) * SQRT_N


@functools.partial(jax.jit, static_argnames=("interpret",))
def _run(query, c_keys, c_prime_keys, ln_weight, ln_bias, interpret=False):
    b, s, h, d = query.shape
    n = b * s * h
    q = query.reshape(n, d)
    ck1 = c_keys.T
    ck2 = c_prime_keys.T
    w = ln_weight.reshape(1, d)
    bias = ln_bias.reshape(1, d)

    block_r = 512
    grid = n // block_r

    gidx, fsc, aux = pl.pallas_call(
        functools.partial(_body, grid=grid),
        grid=(grid,),
        in_specs=[
            pl.BlockSpec((block_r, d), lambda i: (i, 0)),
            pl.BlockSpec((SUB, SQRT_N), lambda i: (0, 0)),
            pl.BlockSpec((SUB, SQRT_N), lambda i: (0, 0)),
            pl.BlockSpec((1, d), lambda i: (0, 0)),
            pl.BlockSpec((1, d), lambda i: (0, 0)),
        ],
        out_specs=[
            pl.BlockSpec((block_r, K), lambda i: (i, 0)),
            pl.BlockSpec((block_r, K), lambda i: (i, 0)),
            pl.BlockSpec((1, 1), lambda i: (0, 0)),
        ],
        out_shape=[
            jax.ShapeDtypeStruct((n, K), jnp.int32),
            jax.ShapeDtypeStruct((n, K), jnp.float32),
            jax.ShapeDtypeStruct((1, 1), jnp.float32),
        ],
        scratch_shapes=[pltpu.VMEM((2, SQRT_N), jnp.float32)],
        interpret=interpret,
    )(q, ck1, ck2, w, bias)

    return (gidx.reshape(b, s, h, K), fsc.reshape(b, s, h, K),
            aux.reshape(()))


def kernel(query, c_keys, c_prime_keys, ln_weight, ln_bias):
    return _run(query, c_keys, c_prime_keys, ln_weight, ln_bias)
